# baseline (speedup 1.0000x reference)
import jax
import jax.numpy as jnp
from jax.experimental import pallas as pl
from jax.experimental.pallas import tpu as pltpu

E = 64
EPAIR = 2
TOP_K = 8
N_GROUP = 8
GSIZE = E // N_GROUP
TOPK_GROUP = 4
D = 1024
F = 512
FS = 2048
FS_CHUNK = 1024
N_SHARED = FS // FS_CHUNK
N_EXP_STEPS = E // EPAIR
BIG = 1e9
ROUTED_SCALING = 2.5


def _route(x, gw, bias):
    logits = jax.lax.dot_general(
        x, gw, (((1,), (1,)), ((), ())), preferred_element_type=jnp.float32)
    probs = jax.nn.sigmoid(logits)
    scores = probs + bias
    T = scores.shape[0]
    iota_g = jax.lax.broadcasted_iota(jnp.int32, (T, N_GROUP), 1)
    gs_cols = []
    for g in range(N_GROUP):
        v = scores[:, g * GSIZE:(g + 1) * GSIZE]
        m1 = jnp.max(v, axis=1, keepdims=True)
        idx1 = jnp.min(jnp.where(v == m1, iota_g, E), axis=1, keepdims=True)
        m2 = jnp.max(jnp.where(iota_g == idx1, -BIG, v), axis=1, keepdims=True)
        gs_cols.append(m1 + m2)
    gscores = jnp.concatenate(gs_cols, axis=1)
    sel_cols = []
    for g in range(N_GROUP):
        sg = gscores[:, g:g + 1]
        gt = (gscores > sg).astype(jnp.float32)
        eq_lo = ((gscores == sg) & (iota_g < g)).astype(jnp.float32)
        rank = jnp.sum(gt + eq_lo, axis=1, keepdims=True)
        sel_cols.append((rank < TOPK_GROUP).astype(jnp.float32))
    smask = jnp.concatenate(
        [jnp.broadcast_to(c, (T, GSIZE)) for c in sel_cols], axis=1)
    masked = jnp.where(smask > 0, scores, 0.0)
    iota_e = jax.lax.broadcasted_iota(jnp.int32, (T, E), 1)
    work = masked
    sel = jnp.zeros((T, E), jnp.float32)
    for _ in range(TOP_K):
        m = jnp.max(work, axis=1, keepdims=True)
        idx = jnp.min(jnp.where(work == m, iota_e, E), axis=1, keepdims=True)
        onehot = (iota_e == idx).astype(jnp.float32)
        sel = sel + onehot
        work = jnp.where(onehot > 0, -BIG, work)
    w = sel * probs
    denom = jnp.sum(w, axis=1, keepdims=True) + 1e-20
    return w * (ROUTED_SCALING / denom)


def _body(x_ref, gw_ref, bias_ref, up_ref, down_ref, su_ref, sd_ref,
          out_ref, w_ref):
    pid = pl.program_id(0)

    @pl.when(pid == 0)
    def _init_router():
        w_ref[...] = _route(x_ref[...], gw_ref[...], bias_ref[...])

    @pl.when(pid < N_SHARED)
    def _shared():
        x = x_ref[...]
        h = jax.lax.dot_general(
            x, su_ref[...], (((1,), (0,)), ((), ())),
            preferred_element_type=jnp.float32)
        h = jnp.maximum(h, 0.0)
        h = h * h
        contrib = jax.lax.dot_general(
            h, sd_ref[...], (((1,), (0,)), ((), ())),
            preferred_element_type=jnp.float32)

        @pl.when(pid == 0)
        def _():
            out_ref[...] = contrib

        @pl.when(pid > 0)
        def _():
            out_ref[...] += contrib

    @pl.when(pid >= N_SHARED)
    def _experts():
        x = x_ref[...].astype(jnp.bfloat16)
        lane = jax.lax.broadcasted_iota(jnp.int32, w_ref.shape, 1)
        epair = pid - N_SHARED
        acc = out_ref[...]
        for j in range(EPAIR):
            h = jax.lax.dot_general(
                x, up_ref[j].astype(jnp.bfloat16), (((1,), (0,)), ((), ())),
                preferred_element_type=jnp.float32)
            h = jnp.maximum(h, 0.0)
            h = h * h
            wcol = jnp.sum(
                jnp.where(lane == epair * EPAIR + j, w_ref[...], 0.0),
                axis=1, keepdims=True)
            h = h * wcol
            acc += jax.lax.dot_general(
                h.astype(jnp.bfloat16), down_ref[j].astype(jnp.bfloat16),
                (((1,), (0,)), ((), ())),
                preferred_element_type=jnp.float32)
        out_ref[...] = acc


def kernel(hidden_states, gate_weight, e_score_correction_bias, expert_up,
           expert_down, shared_up, shared_down):
    orig_shape = hidden_states.shape
    x = hidden_states.reshape(-1, D)
    T = x.shape[0]
    bias = e_score_correction_bias.reshape(1, E)

    grid = (N_SHARED + N_EXP_STEPS,)
    epair_idx = lambda i: jnp.clip(i - N_SHARED, 0, N_EXP_STEPS - 1)
    shared_idx = lambda i: jnp.minimum(i, N_SHARED - 1)
    out = pl.pallas_call(
        _body,
        grid=grid,
        in_specs=[
            pl.BlockSpec((T, D), lambda i: (0, 0)),
            pl.BlockSpec((E, D), lambda i: (0, 0)),
            pl.BlockSpec((1, E), lambda i: (0, 0)),
            pl.BlockSpec((EPAIR, D, F), lambda i: (epair_idx(i), 0, 0)),
            pl.BlockSpec((EPAIR, F, D), lambda i: (epair_idx(i), 0, 0)),
            pl.BlockSpec((D, FS_CHUNK), lambda i: (0, shared_idx(i))),
            pl.BlockSpec((FS_CHUNK, D), lambda i: (shared_idx(i), 0)),
        ],
        out_specs=pl.BlockSpec((T, D), lambda i: (0, 0)),
        out_shape=jax.ShapeDtypeStruct((T, D), jnp.float32),
        scratch_shapes=[pltpu.VMEM((T, E), jnp.float32)],
        compiler_params=pltpu.CompilerParams(
            dimension_semantics=("arbitrary",)),
    )(x, gate_weight, bias, expert_up, expert_down, shared_up, shared_down)
    return out.reshape(orig_shape)


# speedup vs baseline: 1.0046x; 1.0046x over previous
import jax
import jax.numpy as jnp
from jax.experimental import pallas as pl
from jax.experimental.pallas import tpu as pltpu

E = 64
EPAIR = 2
TOP_K = 8
N_GROUP = 8
GSIZE = E // N_GROUP
TOPK_GROUP = 4
D = 1024
F = 512
FS = 2048
FS_CHUNK = 1024
N_SHARED = FS // FS_CHUNK
N_EXP_STEPS = E // EPAIR
BIG = 1e9
ROUTED_SCALING = 2.5


def _route(x, gw, bias):
    logits = jax.lax.dot_general(
        x, gw, (((1,), (1,)), ((), ())), preferred_element_type=jnp.float32)
    probs = jax.nn.sigmoid(logits)
    scores = probs + bias
    T = scores.shape[0]
    iota_g = jax.lax.broadcasted_iota(jnp.int32, (T, N_GROUP), 1)
    gs_cols = []
    for g in range(N_GROUP):
        v = scores[:, g * GSIZE:(g + 1) * GSIZE]
        m1 = jnp.max(v, axis=1, keepdims=True)
        idx1 = jnp.min(jnp.where(v == m1, iota_g, E), axis=1, keepdims=True)
        m2 = jnp.max(jnp.where(iota_g == idx1, -BIG, v), axis=1, keepdims=True)
        gs_cols.append(m1 + m2)
    gscores = jnp.concatenate(gs_cols, axis=1)
    sel_cols = []
    for g in range(N_GROUP):
        sg = gscores[:, g:g + 1]
        gt = (gscores > sg).astype(jnp.float32)
        eq_lo = ((gscores == sg) & (iota_g < g)).astype(jnp.float32)
        rank = jnp.sum(gt + eq_lo, axis=1, keepdims=True)
        sel_cols.append((rank < TOPK_GROUP).astype(jnp.float32))
    smask = jnp.concatenate(
        [jnp.broadcast_to(c, (T, GSIZE)) for c in sel_cols], axis=1)
    masked = jnp.where(smask > 0, scores, 0.0)
    iota_e = jax.lax.broadcasted_iota(jnp.int32, (T, E), 1)
    work = masked
    sel = jnp.zeros((T, E), jnp.float32)
    for _ in range(TOP_K):
        m = jnp.max(work, axis=1, keepdims=True)
        idx = jnp.min(jnp.where(work == m, iota_e, E), axis=1, keepdims=True)
        onehot = (iota_e == idx).astype(jnp.float32)
        sel = sel + onehot
        work = jnp.where(onehot > 0, -BIG, work)
    w = sel * probs
    denom = jnp.sum(w, axis=1, keepdims=True) + 1e-20
    return w * (ROUTED_SCALING / denom)


def _body(x_ref, gw_ref, bias_ref, up_ref, down_ref, su_ref, sd_ref,
          out_ref, w_ref):
    pid = pl.program_id(0)

    @pl.when(pid == 0)
    def _init_router():
        w_ref[...] = _route(x_ref[...], gw_ref[...], bias_ref[...])

    @pl.when(pid < N_SHARED)
    def _shared():
        x = x_ref[...]
        h = jax.lax.dot_general(
            x, su_ref[...], (((1,), (0,)), ((), ())),
            preferred_element_type=jnp.float32)
        h = jnp.maximum(h, 0.0)
        h = h * h
        contrib = jax.lax.dot_general(
            h, sd_ref[...], (((1,), (0,)), ((), ())),
            preferred_element_type=jnp.float32)

        @pl.when(pid == 0)
        def _():
            out_ref[...] = contrib

        @pl.when(pid > 0)
        def _():
            out_ref[...] += contrib

    @pl.when(pid >= N_SHARED)
    def _experts():
        x = x_ref[...]
        lane = jax.lax.broadcasted_iota(jnp.int32, w_ref.shape, 1)
        epair = pid - N_SHARED
        hs = []
        for j in range(EPAIR):
            h = jax.lax.dot_general(
                x, up_ref[j], (((1,), (0,)), ((), ())),
                preferred_element_type=jnp.float32)
            h = jnp.maximum(h, 0.0)
            h = h * h
            wcol = jnp.sum(
                jnp.where(lane == epair * EPAIR + j, w_ref[...], 0.0),
                axis=1, keepdims=True)
            hs.append(h * wcol)
        hcat = jnp.concatenate(hs, axis=1)
        dstack = down_ref[...].reshape(EPAIR * F, D)
        contrib = jax.lax.dot_general(
            hcat, dstack, (((1,), (0,)), ((), ())),
            preferred_element_type=jnp.float32)
        out_ref[...] += contrib


def kernel(hidden_states, gate_weight, e_score_correction_bias, expert_up,
           expert_down, shared_up, shared_down):
    orig_shape = hidden_states.shape
    x = hidden_states.reshape(-1, D)
    T = x.shape[0]
    bias = e_score_correction_bias.reshape(1, E)

    grid = (N_SHARED + N_EXP_STEPS,)
    epair_idx = lambda i: jnp.clip(i - N_SHARED, 0, N_EXP_STEPS - 1)
    shared_idx = lambda i: jnp.minimum(i, N_SHARED - 1)
    out = pl.pallas_call(
        _body,
        grid=grid,
        in_specs=[
            pl.BlockSpec((T, D), lambda i: (0, 0)),
            pl.BlockSpec((E, D), lambda i: (0, 0)),
            pl.BlockSpec((1, E), lambda i: (0, 0)),
            pl.BlockSpec((EPAIR, D, F), lambda i: (epair_idx(i), 0, 0)),
            pl.BlockSpec((EPAIR, F, D), lambda i: (epair_idx(i), 0, 0)),
            pl.BlockSpec((D, FS_CHUNK), lambda i: (0, shared_idx(i))),
            pl.BlockSpec((FS_CHUNK, D), lambda i: (shared_idx(i), 0)),
        ],
        out_specs=pl.BlockSpec((T, D), lambda i: (0, 0)),
        out_shape=jax.ShapeDtypeStruct((T, D), jnp.float32),
        scratch_shapes=[pltpu.VMEM((T, E), jnp.float32)],
        compiler_params=pltpu.CompilerParams(
            dimension_semantics=("arbitrary",)),
    )(x, gate_weight, bias, expert_up, expert_down, shared_up, shared_down)
    return out.reshape(orig_shape)


# speedup vs baseline: 1.0118x; 1.0071x over previous
import jax
import jax.numpy as jnp
from jax.experimental import pallas as pl
from jax.experimental.pallas import tpu as pltpu

E = 64
EPAIR = 2
TOP_K = 8
N_GROUP = 8
GSIZE = E // N_GROUP
TOPK_GROUP = 4
D = 1024
F = 512
FS = 2048
FS_CHUNK = 1024
N_SHARED = FS // FS_CHUNK
N_EXP_STEPS = E // EPAIR
BIG = 1e9
ROUTED_SCALING = 2.5


def _route(x, gw, bias):
    logits = jax.lax.dot_general(
        x, gw, (((1,), (1,)), ((), ())), preferred_element_type=jnp.float32)
    probs = jax.nn.sigmoid(logits)
    scores = probs + bias
    T = scores.shape[0]
    iota_g = jax.lax.broadcasted_iota(jnp.int32, (T, N_GROUP), 1)
    gs_cols = []
    for g in range(N_GROUP):
        v = scores[:, g * GSIZE:(g + 1) * GSIZE]
        m1 = jnp.max(v, axis=1, keepdims=True)
        idx1 = jnp.min(jnp.where(v == m1, iota_g, E), axis=1, keepdims=True)
        m2 = jnp.max(jnp.where(iota_g == idx1, -BIG, v), axis=1, keepdims=True)
        gs_cols.append(m1 + m2)
    gscores = jnp.concatenate(gs_cols, axis=1)
    sel_cols = []
    for g in range(N_GROUP):
        sg = gscores[:, g:g + 1]
        gt = (gscores > sg).astype(jnp.float32)
        eq_lo = ((gscores == sg) & (iota_g < g)).astype(jnp.float32)
        rank = jnp.sum(gt + eq_lo, axis=1, keepdims=True)
        sel_cols.append((rank < TOPK_GROUP).astype(jnp.float32))
    smask = jnp.concatenate(
        [jnp.broadcast_to(c, (T, GSIZE)) for c in sel_cols], axis=1)
    masked = jnp.where(smask > 0, scores, 0.0)
    iota_e = jax.lax.broadcasted_iota(jnp.int32, (T, E), 1)
    work = masked
    sel = jnp.zeros((T, E), jnp.float32)
    for _ in range(TOP_K):
        m = jnp.max(work, axis=1, keepdims=True)
        idx = jnp.min(jnp.where(work == m, iota_e, E), axis=1, keepdims=True)
        onehot = (iota_e == idx).astype(jnp.float32)
        sel = sel + onehot
        work = jnp.where(onehot > 0, -BIG, work)
    w = sel * probs
    denom = jnp.sum(w, axis=1, keepdims=True) + 1e-20
    return w * (ROUTED_SCALING / denom)


def _body(x_ref, gw_ref, bias_ref, up_ref, down_ref, su_ref, sd_ref,
          out_ref, w_ref):
    pid = pl.program_id(0)

    def _shared_contrib():
        x = x_ref[...]
        h = jax.lax.dot_general(
            x, su_ref[...], (((1,), (0,)), ((), ())),
            preferred_element_type=jnp.float32)
        h = jnp.maximum(h, 0.0)
        h = h * h
        return jax.lax.dot_general(
            h, sd_ref[...], (((1,), (0,)), ((), ())),
            preferred_element_type=jnp.float32)

    @pl.when(pid == 0)
    def _shared0_and_router():
        w_ref[...] = _route(x_ref[...], gw_ref[...], bias_ref[...])
        out_ref[...] = _shared_contrib()

    @pl.when((pid > 0) & (pid < N_SHARED))
    def _shared_rest():
        out_ref[...] += _shared_contrib()

    @pl.when(pid >= N_SHARED)
    def _experts():
        x = x_ref[...]
        lane = jax.lax.broadcasted_iota(jnp.int32, w_ref.shape, 1)
        epair = pid - N_SHARED
        hs = []
        for j in range(EPAIR):
            h = jax.lax.dot_general(
                x, up_ref[j], (((1,), (0,)), ((), ())),
                preferred_element_type=jnp.float32)
            h = jnp.maximum(h, 0.0)
            h = h * h
            wcol = jnp.sum(
                jnp.where(lane == epair * EPAIR + j, w_ref[...], 0.0),
                axis=1, keepdims=True)
            hs.append(h * wcol)
        hcat = jnp.concatenate(hs, axis=1)
        dstack = down_ref[...].reshape(EPAIR * F, D)
        contrib = jax.lax.dot_general(
            hcat, dstack, (((1,), (0,)), ((), ())),
            preferred_element_type=jnp.float32)
        out_ref[...] += contrib


def kernel(hidden_states, gate_weight, e_score_correction_bias, expert_up,
           expert_down, shared_up, shared_down):
    orig_shape = hidden_states.shape
    x = hidden_states.reshape(-1, D)
    T = x.shape[0]
    bias = e_score_correction_bias.reshape(1, E)

    grid = (N_SHARED + N_EXP_STEPS,)
    epair_idx = lambda i: jnp.clip(i - N_SHARED, 0, N_EXP_STEPS - 1)
    shared_idx = lambda i: jnp.minimum(i, N_SHARED - 1)
    out = pl.pallas_call(
        _body,
        grid=grid,
        in_specs=[
            pl.BlockSpec((T, D), lambda i: (0, 0)),
            pl.BlockSpec((E, D), lambda i: (0, 0)),
            pl.BlockSpec((1, E), lambda i: (0, 0)),
            pl.BlockSpec((EPAIR, D, F), lambda i: (epair_idx(i), 0, 0)),
            pl.BlockSpec((EPAIR, F, D), lambda i: (epair_idx(i), 0, 0)),
            pl.BlockSpec((D, FS_CHUNK), lambda i: (0, shared_idx(i))),
            pl.BlockSpec((FS_CHUNK, D), lambda i: (shared_idx(i), 0)),
        ],
        out_specs=pl.BlockSpec((T, D), lambda i: (0, 0)),
        out_shape=jax.ShapeDtypeStruct((T, D), jnp.float32),
        scratch_shapes=[pltpu.VMEM((T, E), jnp.float32)],
        compiler_params=pltpu.CompilerParams(
            dimension_semantics=("arbitrary",)),
    )(x, gate_weight, bias, expert_up, expert_down, shared_up, shared_down)
    return out.reshape(orig_shape)
